# baseline (speedup 1.0000x reference)
import functools

import jax
import jax.numpy as jnp
from jax import lax
from jax.experimental import pallas as pl
from jax.experimental.pallas import tpu as pltpu
from jax.experimental.pallas import tpu_sc as plsc

N = 10000
NE = 320000
D = 128
H = 128
C = 64

NC = 2
NS = 16
NW = NC * NS
CHUNK = 128
CPT = 80
EPT = CHUNK * CPT
EP = NW * EPT
NPAD = 10240
RPT = NPAD // NS
ZCH = RPT // CHUNK

RB = 2000
GRID = N // RB


NBUF = 4

_MESH = plsc.VectorSubcoreMesh(core_axis_name="c", subcore_axis_name="s")
_SC_PARAMS = pltpu.CompilerParams(use_tc_tiling_on_sc=False)


@functools.lru_cache(maxsize=None)
def _edge_pass(F):
  @functools.partial(
      pl.kernel,
      out_type=jax.ShapeDtypeStruct((NC, NPAD, F), jnp.float32),
      mesh=_MESH,
      compiler_params=_SC_PARAMS,
      scratch_types=[
          pltpu.VMEM((CPT, CHUNK), jnp.int32),
          pltpu.VMEM((CPT, CHUNK), jnp.int32),
          pltpu.VMEM((NBUF, CHUNK, F), jnp.float32),
          pltpu.VMEM_SHARED((NPAD, F), jnp.float32),
          [pltpu.SemaphoreType.DMA] * NBUF,
          [pltpu.SemaphoreType.DMA] * NBUF,
      ],
  )
  def k(src_hbm, dst_hbm, z_hbm, zrows_hbm, out_hbm,
        sidx, didx, rows, acc, gs, ss):
    cid = lax.axis_index("c")
    sid = lax.axis_index("s")
    wid = cid * NS + sid

    pltpu.sync_copy(src_hbm.at[wid], sidx)
    pltpu.sync_copy(dst_hbm.at[wid], didx)

    base = sid * RPT
    for kk in range(ZCH):
      pltpu.sync_copy(zrows_hbm, acc.at[pl.ds(base + kk * CHUNK, CHUNK)])
    plsc.subcore_barrier()

    def G(c, b):
      return pltpu.make_async_copy(z_hbm.at[sidx.at[c]], rows.at[b], gs[b])

    def S(c, b):
      return pltpu.make_async_copy(rows.at[b], acc.at[didx.at[c]], ss[b])

    for b in range(NBUF):
      G(b, b).start()

    def body(g, carry):
      c0 = NBUF * g
      for b in range(NBUF):
        G(c0 + b, b).wait()
        S(c0 + b, b).start(add=True)
      for b in range(NBUF):
        S(c0 + b, b).wait()
        G(c0 + NBUF + b, b).start()
      return carry

    lax.fori_loop(0, CPT // NBUF - 1, body, 0)
    c0 = CPT - NBUF
    for b in range(NBUF):
      G(c0 + b, b).wait()
      S(c0 + b, b).start(add=True)
    for b in range(NBUF):
      S(c0 + b, b).wait()
    plsc.subcore_barrier()

    for kk in range(ZCH):
      r0 = base + kk * CHUNK
      pltpu.sync_copy(acc.at[pl.ds(r0, CHUNK)], out_hbm.at[cid, pl.ds(r0, CHUNK)])

  return k


DEGW = 16


@functools.lru_cache(maxsize=None)
def _deg_pass():
  @functools.partial(
      pl.kernel,
      out_type=jax.ShapeDtypeStruct((NC, NPAD, DEGW), jnp.float32),
      mesh=_MESH,
      compiler_params=_SC_PARAMS,
      scratch_types=[
          pltpu.VMEM((CPT, CHUNK), jnp.int32),
          pltpu.VMEM((CHUNK, DEGW), jnp.float32),
          pltpu.VMEM_SHARED((NPAD, DEGW), jnp.float32),
          pltpu.SemaphoreType.DMA,
      ],
  )
  def k(dst_hbm, ones_hbm, zrows_hbm, out_hbm, didx, ones_v, acc, sem):
    cid = lax.axis_index("c")
    sid = lax.axis_index("s")
    wid = cid * NS + sid

    pltpu.sync_copy(dst_hbm.at[wid], didx)
    pltpu.sync_copy(ones_hbm, ones_v)
    base = sid * RPT
    for kk in range(ZCH):
      pltpu.sync_copy(zrows_hbm, acc.at[pl.ds(base + kk * CHUNK, CHUNK)])
    plsc.subcore_barrier()

    def S(c):
      return pltpu.make_async_copy(ones_v, acc.at[didx.at[c]], sem)

    def body(g, carry):
      c0 = 8 * g
      for b in range(8):
        S(c0 + b).start(add=True)
      for b in range(8):
        S(c0 + b).wait()
      return carry

    lax.fori_loop(0, CPT // 8, body, 0)
    plsc.subcore_barrier()
    for kk in range(ZCH):
      r0 = base + kk * CHUNK
      pltpu.sync_copy(acc.at[pl.ds(r0, CHUNK)], out_hbm.at[cid, pl.ds(r0, CHUNK)])

  return k


def _dinv(d_ref):
  deg = d_ref[0, :, 0:1] + d_ref[1, :, 0:1] + 1.0
  return lax.rsqrt(deg)


def _tc_a1(x, W1):
  def body(x_ref, w_ref, y_ref):
    y_ref[...] = lax.dot_general(x_ref[...], w_ref[...],
                                 (((1,), (1,)), ((), ())),
                                 preferred_element_type=jnp.float32)

  return pl.pallas_call(
      body,
      grid=(GRID,),
      in_specs=[
          pl.BlockSpec((RB, D), lambda i: (i, 0)),
          pl.BlockSpec((H, D), lambda i: (0, 0)),
      ],
      out_specs=pl.BlockSpec((RB, H), lambda i: (i, 0)),
      out_shape=jax.ShapeDtypeStruct((N, H), jnp.float32),
  )(x, W1)


def _tc_a2(y1, deg2):
  def body(y_ref, d_ref, z_ref):
    z_ref[...] = y_ref[...] * _dinv(d_ref)

  return pl.pallas_call(
      body,
      grid=(GRID,),
      in_specs=[
          pl.BlockSpec((RB, H), lambda i: (i, 0)),
          pl.BlockSpec((NC, RB, DEGW), lambda i: (0, i, 0)),
      ],
      out_specs=pl.BlockSpec((RB, H), lambda i: (i, 0)),
      out_shape=jax.ShapeDtypeStruct((N, H), jnp.float32),
  )(y1, deg2)


def _tc_b(s1a_v, s1b_v, z1, deg2, b1_2d, W2):
  def body(sa_ref, sb_ref, z1_ref, d_ref, b_ref, w_ref, z2_ref):
    dinv = _dinv(d_ref)
    s = jnp.concatenate([sa_ref[0] + sa_ref[1], sb_ref[0] + sb_ref[1]], axis=1)
    pre = (s + z1_ref[...]) * dinv + b_ref[...]
    h = jnp.maximum(pre, 0.0)
    y = lax.dot_general(h, w_ref[...], (((1,), (1,)), ((), ())),
                        preferred_element_type=jnp.float32)
    z2 = y * dinv
    z2_ref[...] = jnp.concatenate([z2, z2], axis=1)

  return pl.pallas_call(
      body,
      grid=(GRID,),
      in_specs=[
          pl.BlockSpec((NC, RB, C), lambda i: (0, i, 0)),
          pl.BlockSpec((NC, RB, C), lambda i: (0, i, 0)),
          pl.BlockSpec((RB, H), lambda i: (i, 0)),
          pl.BlockSpec((NC, RB, DEGW), lambda i: (0, i, 0)),
          pl.BlockSpec((1, H), lambda i: (0, 0)),
          pl.BlockSpec((C, H), lambda i: (0, 0)),
      ],
      out_specs=pl.BlockSpec((RB, 2 * C), lambda i: (i, 0)),
      out_shape=jax.ShapeDtypeStruct((N, 2 * C), jnp.float32),
  )(s1a_v, s1b_v, z1, deg2, b1_2d, W2)


def _tc_c(s2_v, z2cat, deg2, b2_2d):
  def body(s_ref, z2_ref, d_ref, b_ref, o_ref):
    dinv = _dinv(d_ref)
    o = (s_ref[0] + s_ref[1] + z2_ref[:, :C]) * dinv + b_ref[...]
    m = jnp.max(o, axis=1, keepdims=True)
    lse = m + jnp.log(jnp.sum(jnp.exp(o - m), axis=1, keepdims=True))
    o_ref[...] = o - lse

  return pl.pallas_call(
      body,
      grid=(GRID,),
      in_specs=[
          pl.BlockSpec((NC, RB, C), lambda i: (0, i, 0)),
          pl.BlockSpec((RB, 2 * C), lambda i: (i, 0)),
          pl.BlockSpec((NC, RB, DEGW), lambda i: (0, i, 0)),
          pl.BlockSpec((1, C), lambda i: (0, 0)),
      ],
      out_specs=pl.BlockSpec((RB, C), lambda i: (i, 0)),
      out_shape=jax.ShapeDtypeStruct((N, C), jnp.float32),
  )(s2_v, z2cat, deg2, b2_2d)


def kernel(x, edge_index, W1, b1, W2, b2):
  cyc = jnp.arange(EP - NE, dtype=jnp.int32) % (NPAD - N)
  srcp = jnp.concatenate([edge_index[0], cyc]).reshape(NW, CPT, CHUNK)
  dstp = jnp.concatenate([edge_index[1], N + cyc]).reshape(NW, CPT, CHUNK)

  onesd = jnp.ones((CHUNK, DEGW), jnp.float32)
  zdeg = jnp.zeros((CHUNK, DEGW), jnp.float32)
  z128 = jnp.zeros((CHUNK, H), jnp.float32)
  z64 = jnp.zeros((CHUNK, C), jnp.float32)

  sev = srcp * 2
  sod = srcp * 2 + 1

  deg2 = _deg_pass()(dstp, onesd, zdeg)
  y1 = _tc_a1(x, W1)
  z1 = _tc_a2(y1, deg2)
  z1v = z1.reshape(2 * N, C)
  s1a = _edge_pass(C)(sev, dstp, z1v, z64)
  s1b = _edge_pass(C)(sod, dstp, z1v, z64)
  z2cat = _tc_b(s1a, s1b, z1, deg2, b1.reshape(1, H), W2)
  s2 = _edge_pass(C)(sev, dstp, z2cat.reshape(2 * N, C), z64)
  return _tc_c(s2, z2cat, deg2, b2.reshape(1, C))


# speedup vs baseline: 1.0562x; 1.0562x over previous
import functools

import jax
import jax.numpy as jnp
from jax import lax
from jax.experimental import pallas as pl
from jax.experimental.pallas import tpu as pltpu
from jax.experimental.pallas import tpu_sc as plsc

N = 10000
NE = 320000
D = 128
H = 128
C = 64

NC = 2
NS = 16
NW = NC * NS
CHUNK = 128
CPT = 80
EPT = CHUNK * CPT
EP = NW * EPT
NPAD = 10240
RPT = NPAD // NS
ZCH = RPT // CHUNK

RB = 2000
GRID = N // RB


NBUF = 8

_MESH = plsc.VectorSubcoreMesh(core_axis_name="c", subcore_axis_name="s")
_SC_PARAMS = pltpu.CompilerParams(use_tc_tiling_on_sc=False)


@functools.lru_cache(maxsize=None)
def _edge_pass(F):
  @functools.partial(
      pl.kernel,
      out_type=jax.ShapeDtypeStruct((NC, NPAD, F), jnp.float32),
      mesh=_MESH,
      compiler_params=_SC_PARAMS,
      scratch_types=[
          pltpu.VMEM((CPT, CHUNK), jnp.int32),
          pltpu.VMEM((CPT, CHUNK), jnp.int32),
          pltpu.VMEM((NBUF, CHUNK, F), jnp.float32),
          pltpu.VMEM_SHARED((NPAD, F), jnp.float32),
          [pltpu.SemaphoreType.DMA] * NBUF,
          [pltpu.SemaphoreType.DMA] * NBUF,
      ],
  )
  def k(src_hbm, dst_hbm, z_hbm, zrows_hbm, out_hbm,
        sidx, didx, rows, acc, gs, ss):
    cid = lax.axis_index("c")
    sid = lax.axis_index("s")
    wid = cid * NS + sid

    stg_s = pltpu.make_async_copy(src_hbm.at[wid], sidx, gs[0])
    stg_d = pltpu.make_async_copy(dst_hbm.at[wid], didx, gs[1])
    stg_s.start()
    stg_d.start()

    base = sid * RPT
    for kk in range(ZCH):
      pltpu.sync_copy(zrows_hbm, acc.at[pl.ds(base + kk * CHUNK, CHUNK)])
    stg_s.wait()
    stg_d.wait()
    plsc.subcore_barrier()

    def G(c, b):
      return pltpu.make_async_copy(z_hbm.at[sidx.at[c]], rows.at[b], gs[b])

    def S(c, b):
      return pltpu.make_async_copy(rows.at[b], acc.at[didx.at[c]], ss[b])

    for b in range(NBUF):
      G(b, b).start()

    def body(g, carry):
      c0 = NBUF * g
      for b in range(NBUF):
        G(c0 + b, b).wait()
        S(c0 + b, b).start(add=True)
      for b in range(NBUF):
        S(c0 + b, b).wait()
        G(c0 + NBUF + b, b).start()
      return carry

    lax.fori_loop(0, CPT // NBUF - 1, body, 0)
    c0 = CPT - NBUF
    for b in range(NBUF):
      G(c0 + b, b).wait()
      S(c0 + b, b).start(add=True)
    for b in range(NBUF):
      S(c0 + b, b).wait()
    plsc.subcore_barrier()

    wbs = []
    for kk in range(ZCH):
      r0 = base + kk * CHUNK
      wb = pltpu.make_async_copy(acc.at[pl.ds(r0, CHUNK)],
                                 out_hbm.at[cid, pl.ds(r0, CHUNK)], gs[kk % NBUF])
      wb.start()
      wbs.append(wb)
    for wb in wbs:
      wb.wait()

  return k


DEGW = 16


@functools.lru_cache(maxsize=None)
def _deg_pass():
  @functools.partial(
      pl.kernel,
      out_type=jax.ShapeDtypeStruct((NC, NPAD, DEGW), jnp.float32),
      mesh=_MESH,
      compiler_params=_SC_PARAMS,
      scratch_types=[
          pltpu.VMEM((CPT, CHUNK), jnp.int32),
          pltpu.VMEM((CHUNK, DEGW), jnp.float32),
          pltpu.VMEM_SHARED((NPAD, DEGW), jnp.float32),
          pltpu.SemaphoreType.DMA,
      ],
  )
  def k(dst_hbm, ones_hbm, zrows_hbm, out_hbm, didx, ones_v, acc, sem):
    cid = lax.axis_index("c")
    sid = lax.axis_index("s")
    wid = cid * NS + sid

    pltpu.sync_copy(dst_hbm.at[wid], didx)
    pltpu.sync_copy(ones_hbm, ones_v)
    base = sid * RPT
    for kk in range(ZCH):
      pltpu.sync_copy(zrows_hbm, acc.at[pl.ds(base + kk * CHUNK, CHUNK)])
    plsc.subcore_barrier()

    def S(c):
      return pltpu.make_async_copy(ones_v, acc.at[didx.at[c]], sem)

    def body(g, carry):
      c0 = 8 * g
      for b in range(8):
        S(c0 + b).start(add=True)
      for b in range(8):
        S(c0 + b).wait()
      return carry

    lax.fori_loop(0, CPT // 8, body, 0)
    plsc.subcore_barrier()
    for kk in range(ZCH):
      r0 = base + kk * CHUNK
      pltpu.sync_copy(acc.at[pl.ds(r0, CHUNK)], out_hbm.at[cid, pl.ds(r0, CHUNK)])

  return k


def _dinv(d_ref):
  deg = d_ref[0, :, 0:1] + d_ref[1, :, 0:1] + 1.0
  return lax.rsqrt(deg)


def _tc_a(x, W1, deg2):
  def body(x_ref, w_ref, d_ref, z_ref):
    dinv = _dinv(d_ref)
    y = lax.dot_general(x_ref[...], w_ref[...], (((1,), (1,)), ((), ())),
                        preferred_element_type=jnp.float32)
    z_ref[...] = y * dinv

  return pl.pallas_call(
      body,
      grid=(GRID,),
      in_specs=[
          pl.BlockSpec((RB, D), lambda i: (i, 0)),
          pl.BlockSpec((H, D), lambda i: (0, 0)),
          pl.BlockSpec((NC, RB, DEGW), lambda i: (0, i, 0)),
      ],
      out_specs=pl.BlockSpec((RB, H), lambda i: (i, 0)),
      out_shape=jax.ShapeDtypeStruct((N, H), jnp.float32),
  )(x, W1, deg2)


def _tc_b(s1a_v, s1b_v, z1, deg2, b1_2d, W2):
  def body(sa_ref, sb_ref, z1_ref, d_ref, b_ref, w_ref, z2_ref):
    dinv = _dinv(d_ref)
    s = jnp.concatenate([sa_ref[0] + sa_ref[1], sb_ref[0] + sb_ref[1]], axis=1)
    pre = (s + z1_ref[...]) * dinv + b_ref[...]
    h = jnp.maximum(pre, 0.0)
    y = lax.dot_general(h, w_ref[...], (((1,), (1,)), ((), ())),
                        preferred_element_type=jnp.float32)
    z2 = y * dinv
    z2_ref[...] = jnp.concatenate([z2, z2], axis=1)

  return pl.pallas_call(
      body,
      grid=(GRID,),
      in_specs=[
          pl.BlockSpec((NC, RB, C), lambda i: (0, i, 0)),
          pl.BlockSpec((NC, RB, C), lambda i: (0, i, 0)),
          pl.BlockSpec((RB, H), lambda i: (i, 0)),
          pl.BlockSpec((NC, RB, DEGW), lambda i: (0, i, 0)),
          pl.BlockSpec((1, H), lambda i: (0, 0)),
          pl.BlockSpec((C, H), lambda i: (0, 0)),
      ],
      out_specs=pl.BlockSpec((RB, 2 * C), lambda i: (i, 0)),
      out_shape=jax.ShapeDtypeStruct((N, 2 * C), jnp.float32),
  )(s1a_v, s1b_v, z1, deg2, b1_2d, W2)


def _tc_c(s2_v, z2cat, deg2, b2_2d):
  def body(s_ref, z2_ref, d_ref, b_ref, o_ref):
    dinv = _dinv(d_ref)
    o = (s_ref[0] + s_ref[1] + z2_ref[:, :C]) * dinv + b_ref[...]
    m = jnp.max(o, axis=1, keepdims=True)
    lse = m + jnp.log(jnp.sum(jnp.exp(o - m), axis=1, keepdims=True))
    o_ref[...] = o - lse

  return pl.pallas_call(
      body,
      grid=(GRID,),
      in_specs=[
          pl.BlockSpec((NC, RB, C), lambda i: (0, i, 0)),
          pl.BlockSpec((RB, 2 * C), lambda i: (i, 0)),
          pl.BlockSpec((NC, RB, DEGW), lambda i: (0, i, 0)),
          pl.BlockSpec((1, C), lambda i: (0, 0)),
      ],
      out_specs=pl.BlockSpec((RB, C), lambda i: (i, 0)),
      out_shape=jax.ShapeDtypeStruct((N, C), jnp.float32),
  )(s2_v, z2cat, deg2, b2_2d)


def kernel(x, edge_index, W1, b1, W2, b2):
  cyc = jnp.arange(EP - NE, dtype=jnp.int32) % (NPAD - N)
  srcp = jnp.concatenate([edge_index[0], cyc]).reshape(NW, CPT, CHUNK)
  dstp = jnp.concatenate([edge_index[1], N + cyc]).reshape(NW, CPT, CHUNK)

  onesd = jnp.ones((CHUNK, DEGW), jnp.float32)
  zdeg = jnp.zeros((CHUNK, DEGW), jnp.float32)
  z128 = jnp.zeros((CHUNK, H), jnp.float32)
  z64 = jnp.zeros((CHUNK, C), jnp.float32)

  sev = srcp * 2
  sod = srcp * 2 + 1

  deg2 = _deg_pass()(dstp, onesd, zdeg)
  z1 = _tc_a(x, W1, deg2)
  z1v = z1.reshape(2 * N, C)
  s1a = _edge_pass(C)(sev, dstp, z1v, z64)
  s1b = _edge_pass(C)(sod, dstp, z1v, z64)
  z2cat = _tc_b(s1a, s1b, z1, deg2, b1.reshape(1, H), W2)
  s2 = _edge_pass(C)(sev, dstp, z2cat.reshape(2 * N, C), z64)
  return _tc_c(s2, z2cat, deg2, b2.reshape(1, C))
